# baseline (speedup 1.0000x reference)
import functools

import jax
import jax.numpy as jnp
from jax import lax
from jax.experimental import pallas as pl
from jax.experimental.pallas import tpu as pltpu
from jax.experimental.pallas import tpu_sc as plsc

_N = 10000
_K = 16
_DIM = 128
_H = 64
_BN = 400

_NW = 32
_NCHUNK = 125


def _sigmoid(x):
    return 0.5 * jnp.tanh(0.5 * x) + 0.5


def _bilstm_mean(getpx, wh, rows, nsteps):
    col = lax.broadcasted_iota(jnp.int32, (1, 8 * _H), 1)
    fmask = (col % (2 * _H)) < _H
    zero = jnp.zeros((rows, 2 * _H), jnp.float32)
    h, c, s = zero, zero, zero
    for t in range(nsteps):
        px = jnp.where(fmask, getpx(t), getpx(nsteps - 1 - t))
        g = px + jnp.dot(h, wh, preferred_element_type=jnp.float32)
        sif = _sigmoid(g[:, :4 * _H])
        gg = jnp.tanh(g[:, 4 * _H:6 * _H])
        o = _sigmoid(g[:, 6 * _H:])
        c = sif[:, 2 * _H:] * c + sif[:, :2 * _H] * gg
        h = o * jnp.tanh(c)
        s = s + h
    return s * (1.0 / nsteps)


def _content_body(x, wiT, whT, bias, out):
    bn = x.shape[2]
    xs = x[...].reshape(3 * bn, _DIM)
    px = (jnp.dot(xs, wiT[...].reshape(_DIM, 8 * _H),
                  preferred_element_type=jnp.float32)
          + bias[...].reshape(1, 8 * _H))
    res = _bilstm_mean(lambda t: px[t * bn:(t + 1) * bn],
                       whT[...].reshape(2 * _H, 8 * _H), bn, 3)
    out[...] = res.reshape(1, bn, _DIM)


def _neigh_body(m, wiT, whT, bias, out):
    bn = m.shape[1]
    mv = m[...].reshape(_K * bn, _DIM)
    px = jnp.dot(mv, wiT[...], preferred_element_type=jnp.float32) + bias[...]
    out[...] = _bilstm_mean(lambda t: px[t * bn:(t + 1) * bn], whT[...], bn, _K)


def _atten_body(dh, ng, w, b, out):
    bn = dh.shape[1]
    d = dh[...].reshape(bn, _DIM)
    ngv = ng[...]
    e0, e1, e2 = ngv[0, 0], ngv[0, 1], ngv[0, 2]
    wv = w[...].reshape(1, 2 * _DIM)
    w1 = wv[0:1, :_DIM]
    w2 = wv[0:1, _DIM:]
    bias = b[...].reshape(1, 1)
    u = jnp.sum(d * w1, axis=1, keepdims=True) + bias
    s0 = u + jnp.sum(e0 * w2, axis=1, keepdims=True)
    s1 = u + jnp.sum(e1 * w2, axis=1, keepdims=True)
    s2 = u + jnp.sum(e2 * w2, axis=1, keepdims=True)
    s3 = u + jnp.sum(d * w2, axis=1, keepdims=True)
    s = jnp.concatenate([s0, s1, s2, s3], axis=1)
    s = jnp.where(s >= 0, s, 0.01 * s)
    s = s - jnp.max(s, axis=1, keepdims=True)
    e = jnp.exp(s)
    a = e / jnp.sum(e, axis=1, keepdims=True)
    res = (a[:, 0:1] * e0 + a[:, 1:2] * e1 + a[:, 2:3] * e2
           + a[:, 3:4] * d)
    out[...] = res.reshape(1, bn, _DIM)


def _row_spec(bn, d):
    return pl.BlockSpec((bn, d), lambda i: (i, 0))


def _full2(shape):
    return pl.BlockSpec(shape, lambda i: (0, 0))


def _ilv(af, ab):
    x = af.shape[0]
    return jnp.stack([af.reshape(x, 4, _H), ab.reshape(x, 4, _H)],
                     axis=2).reshape(x, 8 * _H)


def _prep_lstm(lstm):
    (wif, whf, bf), (wib, whb, bb) = lstm
    wiT = _ilv(wif.T, wib.T)
    z = jnp.zeros((_H, 4 * _H), jnp.float32)
    whT = jnp.concatenate([_ilv(whf.T, z), _ilv(z, whb.T)], axis=0)
    bias = _ilv(bf.reshape(1, -1), bb.reshape(1, -1))
    return wiT, whT, bias


def _content_call(x_all, lstms):
    preps = [_prep_lstm(l) for l in lstms]
    wiT = jnp.stack([p[0] for p in preps])
    whT = jnp.stack([p[1] for p in preps])
    bias = jnp.stack([p[2] for p in preps])
    return pl.pallas_call(
        _content_body,
        grid=(3, _N // _BN),
        in_specs=[pl.BlockSpec((1, 3, _BN, _DIM), lambda t, i: (t, 0, i, 0)),
                  pl.BlockSpec((1, _DIM, 8 * _H), lambda t, i: (t, 0, 0)),
                  pl.BlockSpec((1, 2 * _H, 8 * _H), lambda t, i: (t, 0, 0)),
                  pl.BlockSpec((1, 1, 8 * _H), lambda t, i: (t, 0, 0))],
        out_specs=pl.BlockSpec((1, _BN, _DIM), lambda t, i: (t, i, 0)),
        out_shape=jax.ShapeDtypeStruct((3, _N, _DIM), jnp.float32),
    )(x_all, wiT, whT, bias)


def _neigh_call(mail, lstm):
    wiT, whT, bias = _prep_lstm(lstm)
    rows = mail.shape[1]
    grid = (rows // _BN,)
    return pl.pallas_call(
        _neigh_body,
        grid=grid,
        in_specs=[pl.BlockSpec((_K, _BN, _DIM), lambda i: (0, i, 0)),
                  _full2((_DIM, 8 * _H)), _full2((2 * _H, 8 * _H)),
                  _full2((1, 8 * _H))],
        out_specs=_row_spec(_BN, _DIM),
        out_shape=jax.ShapeDtypeStruct((rows, _DIM), jnp.float32),
    )(mail, wiT, whT, bias)


def _atten_call(dh_all, ng_all, attens):
    w = jnp.stack([a[0] for a in attens])
    b = jnp.stack([a[1].reshape(1, 1) for a in attens])
    return pl.pallas_call(
        _atten_body,
        grid=(3, _N // _BN),
        in_specs=[pl.BlockSpec((1, _BN, _DIM), lambda t, i: (t, i, 0)),
                  pl.BlockSpec((1, 3, _BN, _DIM), lambda t, i: (t, 0, i, 0)),
                  pl.BlockSpec((1, 1, 2 * _DIM), lambda t, i: (t, 0, 0)),
                  pl.BlockSpec((1, 1, 1), lambda t, i: (t, 0, 0))],
        out_specs=pl.BlockSpec((1, _BN, _DIM), lambda t, i: (t, i, 0)),
        out_shape=jax.ShapeDtypeStruct((3, _N, _DIM), jnp.float32),
    )(dh_all, ng_all, w, b)


@functools.cache
def _make_sc_gather(nrows, chunk):
    assert nrows == _NW * _NCHUNK * chunk and chunk % 8 == 0 and chunk <= 128
    b_per_w = nrows // _NW
    mesh = plsc.VectorSubcoreMesh(core_axis_name="c", subcore_axis_name="s")

    @functools.partial(
        pl.kernel,
        out_type=jax.ShapeDtypeStruct((nrows, _DIM), jnp.float32),
        mesh=mesh,
        scratch_types=[
            pltpu.VMEM((_NCHUNK, chunk), jnp.int32),
            pltpu.VMEM((chunk, _DIM), jnp.float32),
            pltpu.VMEM((chunk, _DIM), jnp.float32),
            pltpu.VMEM((chunk, _DIM), jnp.float32),
            pltpu.SemaphoreType.DMA,
            pltpu.SemaphoreType.DMA,
            pltpu.SemaphoreType.DMA,
            pltpu.SemaphoreType.DMA,
            pltpu.SemaphoreType.DMA,
            pltpu.SemaphoreType.DMA,
        ],
    )
    def gather_k(table_hbm, idx_hbm, out_hbm, idx_v,
                 r0, r1, r2, gs0, gs1, gs2, ws0, ws1, ws2):
        wid = lax.axis_index("s") * 2 + lax.axis_index("c")
        base = pl.multiple_of(wid * b_per_w, 8)
        pltpu.sync_copy(idx_hbm.at[wid], idx_v)

        bufs = ((r0, gs0, ws0), (r1, gs1, ws1), (r2, gs2, ws2))

        def gath(c, j):
            r, gs, _ = bufs[j]
            return pltpu.make_async_copy(table_hbm.at[idx_v.at[c]], r, gs)

        def wrb(c, j):
            r, _, ws = bufs[j]
            off = pl.multiple_of(base + c * chunk, 8)
            return pltpu.make_async_copy(r, out_hbm.at[pl.ds(off, chunk)], ws)

        gath(0, 0).start()
        gath(1, 1).start()
        gath(2, 2).start()
        gath(0, 0).wait()
        wrb(0, 0).start()

        def body(i, carry):
            c0 = 3 + 3 * i
            for j in range(3):
                c = c0 + j
                wrb(c - 3, j).wait()
                gath(c, j).start()
                gath(c - 2, (j + 1) % 3).wait()
                wrb(c - 2, (j + 1) % 3).start()
            return carry

        lax.fori_loop(0, (_NCHUNK - 5) // 3, body, 0)

        nc = _NCHUNK
        wrb(nc - 5, (nc - 5) % 3).wait()
        gath(nc - 2, (nc - 2) % 3).start()
        gath(nc - 4, (nc - 4) % 3).wait()
        wrb(nc - 4, (nc - 4) % 3).start()
        wrb(nc - 4, (nc - 4) % 3).wait()
        gath(nc - 1, (nc - 1) % 3).start()
        gath(nc - 3, (nc - 3) % 3).wait()
        wrb(nc - 3, (nc - 3) % 3).start()
        gath(nc - 2, (nc - 2) % 3).wait()
        wrb(nc - 2, (nc - 2) % 3).start()
        gath(nc - 1, (nc - 1) % 3).wait()
        wrb(nc - 1, (nc - 1) % 3).start()
        wrb(nc - 3, (nc - 3) % 3).wait()
        wrb(nc - 2, (nc - 2) % 3).wait()
        wrb(nc - 1, (nc - 1) % 3).wait()

    return gather_k


def _sc_gather(table, idx):
    nrows = idx.shape[0]
    chunk = nrows // (_NW * _NCHUNK)
    idx3 = idx.reshape(_NW, _NCHUNK, chunk)
    return _make_sc_gather(nrows, chunk)(table, idx3)


def kernel(x_a_0, x_a_1, x_a_2, x_b_0, x_b_1, x_b_2, x_c_0, x_c_1, x_c_2,
           nbr_a_a, nbr_a_b, nbr_a_c, nbr_b_a, nbr_b_b, nbr_b_c,
           nbr_c_a, nbr_c_b, nbr_c_c,
           content_lstm_a, content_lstm_b, content_lstm_c,
           neigh_lstm_a, neigh_lstm_b, neigh_lstm_c,
           atten_a, atten_b, atten_c):
    xs = {"a": (x_a_0, x_a_1, x_a_2),
          "b": (x_b_0, x_b_1, x_b_2),
          "c": (x_c_0, x_c_1, x_c_2)}
    content_lstm = {"a": content_lstm_a, "b": content_lstm_b, "c": content_lstm_c}
    neigh_lstm = {"a": neigh_lstm_a, "b": neigh_lstm_b, "c": neigh_lstm_c}
    atten = {"a": atten_a, "b": atten_b, "c": atten_c}
    nbr = {("a", "a"): nbr_a_a, ("a", "b"): nbr_a_b, ("a", "c"): nbr_a_c,
           ("b", "a"): nbr_b_a, ("b", "b"): nbr_b_b, ("b", "c"): nbr_b_c,
           ("c", "a"): nbr_c_a, ("c", "b"): nbr_c_b, ("c", "c"): nbr_c_c}
    types = ("a", "b", "c")

    x_all = jnp.stack([jnp.stack(xs[t]) for t in types])
    content_all = _content_call(x_all, [content_lstm[t] for t in types])
    content = {t: content_all[ti] for ti, t in enumerate(types)}

    mails = {}
    for si, s in enumerate(types):
        if si == 0:
            idx1 = nbr[(types[0], s)].T.reshape(-1)
            idx2 = jnp.concatenate([nbr[(d, s)] for d in types[1:]],
                                   axis=0).T.reshape(-1)
            mails[s] = (
                _sc_gather(content[s], idx1).reshape(_K, _N, _DIM),
                _sc_gather(content[s], idx2).reshape(_K, 2 * _N, _DIM),
            )
        else:
            idx = jnp.concatenate([nbr[(d, s)] for d in types],
                                  axis=0).T.reshape(-1)
            mails[s] = (_sc_gather(content[s], idx).reshape(_K, 3 * _N, _DIM),)
    neigh = {}
    for s in types:
        pieces = [_neigh_call(mm, neigh_lstm[s]) for mm in mails[s]]
        neigh[s] = (jnp.concatenate(pieces, axis=0)
                    if len(pieces) > 1 else pieces[0])

    ng_all = jnp.stack([
        jnp.stack([neigh[s][di * _N:(di + 1) * _N] for s in types])
        for di in range(3)])
    out_all = _atten_call(content_all, ng_all, [atten[t] for t in types])
    return tuple(out_all[di] for di in range(3))


# speedup vs baseline: 1.1536x; 1.1536x over previous
import functools

import jax
import jax.numpy as jnp
from jax import lax
from jax.experimental import pallas as pl
from jax.experimental.pallas import tpu as pltpu
from jax.experimental.pallas import tpu_sc as plsc

_N = 10000
_K = 16
_DIM = 128
_H = 64
_BN = 400

_NW = 32
_NCHUNK = 125


def _sigmoid(x):
    return 0.5 * jnp.tanh(0.5 * x) + 0.5


def _bilstm_mean(getpx, wh, rows, nsteps):
    col = lax.broadcasted_iota(jnp.int32, (1, 8 * _H), 1)
    fmask = (col % (2 * _H)) < _H
    zero = jnp.zeros((rows, 2 * _H), jnp.float32)
    h, c, s = zero, zero, zero
    for t in range(nsteps):
        px = jnp.where(fmask, getpx(t), getpx(nsteps - 1 - t))
        g = px + jnp.dot(h, wh, preferred_element_type=jnp.float32)
        sif = _sigmoid(g[:, :4 * _H])
        gg = jnp.tanh(g[:, 4 * _H:6 * _H])
        o = _sigmoid(g[:, 6 * _H:])
        c = sif[:, 2 * _H:] * c + sif[:, :2 * _H] * gg
        h = o * jnp.tanh(c)
        s = s + h
    return s * (1.0 / nsteps)


def _content_body(x0, x1, x2, wiT, whT, bias, out):
    bn = x0.shape[0]
    xs = jnp.concatenate([x0[...], x1[...], x2[...]], axis=0)
    px = jnp.dot(xs, wiT[...], preferred_element_type=jnp.float32) + bias[...]
    out[...] = _bilstm_mean(lambda t: px[t * bn:(t + 1) * bn], whT[...], bn, 3)


def _neigh_body(m, wiT, whT, bias, out):
    bn = m.shape[1]
    mv = m[...].reshape(_K * bn, _DIM)
    px = jnp.dot(mv, wiT[...], preferred_element_type=jnp.float32) + bias[...]
    out[...] = _bilstm_mean(lambda t: px[t * bn:(t + 1) * bn], whT[...], bn, _K)


def _atten_body(dh, n0, n1, n2, w, b, out):
    d = dh[...]
    e0, e1, e2 = n0[...], n1[...], n2[...]
    wv = w[...]
    w1 = wv[0:1, :_DIM]
    w2 = wv[0:1, _DIM:]
    bias = b[...]
    u = jnp.sum(d * w1, axis=1, keepdims=True) + bias
    s0 = u + jnp.sum(e0 * w2, axis=1, keepdims=True)
    s1 = u + jnp.sum(e1 * w2, axis=1, keepdims=True)
    s2 = u + jnp.sum(e2 * w2, axis=1, keepdims=True)
    s3 = u + jnp.sum(d * w2, axis=1, keepdims=True)
    s = jnp.concatenate([s0, s1, s2, s3], axis=1)
    s = jnp.where(s >= 0, s, 0.01 * s)
    s = s - jnp.max(s, axis=1, keepdims=True)
    e = jnp.exp(s)
    a = e / jnp.sum(e, axis=1, keepdims=True)
    out[...] = (a[:, 0:1] * e0 + a[:, 1:2] * e1 + a[:, 2:3] * e2
                + a[:, 3:4] * d)


def _row_spec(bn, d):
    return pl.BlockSpec((bn, d), lambda i: (i, 0))


def _full2(shape):
    return pl.BlockSpec(shape, lambda i: (0, 0))


def _ilv(af, ab):
    x = af.shape[0]
    return jnp.stack([af.reshape(x, 4, _H), ab.reshape(x, 4, _H)],
                     axis=2).reshape(x, 8 * _H)


def _prep_lstm(lstm):
    (wif, whf, bf), (wib, whb, bb) = lstm
    wiT = _ilv(wif.T, wib.T)
    z = jnp.zeros((_H, 4 * _H), jnp.float32)
    whT = jnp.concatenate([_ilv(whf.T, z), _ilv(z, whb.T)], axis=0)
    bias = _ilv(bf.reshape(1, -1), bb.reshape(1, -1))
    return wiT, whT, bias


def _content_call(x0, x1, x2, lstm):
    wiT, whT, bias = _prep_lstm(lstm)
    grid = (_N // _BN,)
    rs = _row_spec(_BN, _DIM)
    return pl.pallas_call(
        _content_body,
        grid=grid,
        in_specs=[rs, rs, rs,
                  _full2((_DIM, 8 * _H)), _full2((2 * _H, 8 * _H)),
                  _full2((1, 8 * _H))],
        out_specs=rs,
        out_shape=jax.ShapeDtypeStruct((_N, _DIM), jnp.float32),
    )(x0, x1, x2, wiT, whT, bias)


def _neigh_call(mail, lstm):
    wiT, whT, bias = _prep_lstm(lstm)
    rows = mail.shape[1]
    grid = (rows // _BN,)
    return pl.pallas_call(
        _neigh_body,
        grid=grid,
        in_specs=[pl.BlockSpec((_K, _BN, _DIM), lambda i: (0, i, 0)),
                  _full2((_DIM, 8 * _H)), _full2((2 * _H, 8 * _H)),
                  _full2((1, 8 * _H))],
        out_specs=_row_spec(_BN, _DIM),
        out_shape=jax.ShapeDtypeStruct((rows, _DIM), jnp.float32),
    )(mail, wiT, whT, bias)


def _atten_call(dh, n0, n1, n2, att):
    w, b = att
    grid = (_N // _BN,)
    rs = _row_spec(_BN, _DIM)
    return pl.pallas_call(
        _atten_body,
        grid=grid,
        in_specs=[rs, rs, rs, rs, _full2((1, 2 * _DIM)), _full2((1, 1))],
        out_specs=rs,
        out_shape=jax.ShapeDtypeStruct((_N, _DIM), jnp.float32),
    )(dh, n0, n1, n2, w, b.reshape(1, 1))


@functools.cache
def _make_sc_gather(nrows, chunk):
    assert nrows == _NW * _NCHUNK * chunk and chunk % 8 == 0 and chunk <= 128
    b_per_w = nrows // _NW
    mesh = plsc.VectorSubcoreMesh(core_axis_name="c", subcore_axis_name="s")

    @functools.partial(
        pl.kernel,
        out_type=jax.ShapeDtypeStruct((nrows, _DIM), jnp.float32),
        mesh=mesh,
        scratch_types=[
            pltpu.VMEM((_NCHUNK, chunk), jnp.int32),
            pltpu.VMEM((chunk, _DIM), jnp.float32),
            pltpu.VMEM((chunk, _DIM), jnp.float32),
            pltpu.VMEM((chunk, _DIM), jnp.float32),
            pltpu.SemaphoreType.DMA,
            pltpu.SemaphoreType.DMA,
            pltpu.SemaphoreType.DMA,
            pltpu.SemaphoreType.DMA,
            pltpu.SemaphoreType.DMA,
            pltpu.SemaphoreType.DMA,
        ],
    )
    def gather_k(table_hbm, idx_hbm, out_hbm, idx_v,
                 r0, r1, r2, gs0, gs1, gs2, ws0, ws1, ws2):
        wid = lax.axis_index("s") * 2 + lax.axis_index("c")
        base = pl.multiple_of(wid * b_per_w, 8)
        pltpu.sync_copy(idx_hbm.at[wid], idx_v)

        bufs = ((r0, gs0, ws0), (r1, gs1, ws1), (r2, gs2, ws2))

        def gath(c, j):
            r, gs, _ = bufs[j]
            return pltpu.make_async_copy(table_hbm.at[idx_v.at[c]], r, gs)

        def wrb(c, j):
            r, _, ws = bufs[j]
            off = pl.multiple_of(base + c * chunk, 8)
            return pltpu.make_async_copy(r, out_hbm.at[pl.ds(off, chunk)], ws)

        gath(0, 0).start()
        gath(1, 1).start()
        gath(2, 2).start()
        gath(0, 0).wait()
        wrb(0, 0).start()

        def body(i, carry):
            c0 = 3 + 3 * i
            for j in range(3):
                c = c0 + j
                wrb(c - 3, j).wait()
                gath(c, j).start()
                gath(c - 2, (j + 1) % 3).wait()
                wrb(c - 2, (j + 1) % 3).start()
            return carry

        lax.fori_loop(0, (_NCHUNK - 5) // 3, body, 0)

        nc = _NCHUNK
        wrb(nc - 5, (nc - 5) % 3).wait()
        gath(nc - 2, (nc - 2) % 3).start()
        gath(nc - 4, (nc - 4) % 3).wait()
        wrb(nc - 4, (nc - 4) % 3).start()
        wrb(nc - 4, (nc - 4) % 3).wait()
        gath(nc - 1, (nc - 1) % 3).start()
        gath(nc - 3, (nc - 3) % 3).wait()
        wrb(nc - 3, (nc - 3) % 3).start()
        gath(nc - 2, (nc - 2) % 3).wait()
        wrb(nc - 2, (nc - 2) % 3).start()
        gath(nc - 1, (nc - 1) % 3).wait()
        wrb(nc - 1, (nc - 1) % 3).start()
        wrb(nc - 3, (nc - 3) % 3).wait()
        wrb(nc - 2, (nc - 2) % 3).wait()
        wrb(nc - 1, (nc - 1) % 3).wait()

    return gather_k


def _sc_gather(table, idx):
    nrows = idx.shape[0]
    chunk = nrows // (_NW * _NCHUNK)
    idx3 = idx.reshape(_NW, _NCHUNK, chunk)
    return _make_sc_gather(nrows, chunk)(table, idx3)


def kernel(x_a_0, x_a_1, x_a_2, x_b_0, x_b_1, x_b_2, x_c_0, x_c_1, x_c_2,
           nbr_a_a, nbr_a_b, nbr_a_c, nbr_b_a, nbr_b_b, nbr_b_c,
           nbr_c_a, nbr_c_b, nbr_c_c,
           content_lstm_a, content_lstm_b, content_lstm_c,
           neigh_lstm_a, neigh_lstm_b, neigh_lstm_c,
           atten_a, atten_b, atten_c):
    xs = {"a": (x_a_0, x_a_1, x_a_2),
          "b": (x_b_0, x_b_1, x_b_2),
          "c": (x_c_0, x_c_1, x_c_2)}
    content_lstm = {"a": content_lstm_a, "b": content_lstm_b, "c": content_lstm_c}
    neigh_lstm = {"a": neigh_lstm_a, "b": neigh_lstm_b, "c": neigh_lstm_c}
    atten = {"a": atten_a, "b": atten_b, "c": atten_c}
    nbr = {("a", "a"): nbr_a_a, ("a", "b"): nbr_a_b, ("a", "c"): nbr_a_c,
           ("b", "a"): nbr_b_a, ("b", "b"): nbr_b_b, ("b", "c"): nbr_b_c,
           ("c", "a"): nbr_c_a, ("c", "b"): nbr_c_b, ("c", "c"): nbr_c_c}
    types = ("a", "b", "c")

    content = {t: _content_call(*xs[t], content_lstm[t]) for t in types}

    mails = {}
    for si, s in enumerate(types):
        if si == 0:
            idx1 = nbr[(types[0], s)].T.reshape(-1)
            idx2 = jnp.concatenate([nbr[(d, s)] for d in types[1:]],
                                   axis=0).T.reshape(-1)
            mails[s] = (
                _sc_gather(content[s], idx1).reshape(_K, _N, _DIM),
                _sc_gather(content[s], idx2).reshape(_K, 2 * _N, _DIM),
            )
        else:
            idx = jnp.concatenate([nbr[(d, s)] for d in types],
                                  axis=0).T.reshape(-1)
            mails[s] = (_sc_gather(content[s], idx).reshape(_K, 3 * _N, _DIM),)
    neigh = {}
    for s in types:
        pieces = [_neigh_call(mm, neigh_lstm[s]) for mm in mails[s]]
        neigh[s] = (jnp.concatenate(pieces, axis=0)
                    if len(pieces) > 1 else pieces[0])

    outs = []
    for di, d in enumerate(types):
        ngs = [neigh[s][di * _N:(di + 1) * _N] for s in types]
        outs.append(_atten_call(content[d], ngs[0], ngs[1], ngs[2], atten[d]))
    return tuple(outs)
